# Initial kernel scaffold; baseline (speedup 1.0000x reference)
import jax
import jax.numpy as jnp
from jax.experimental import pallas as pl


def kernel(H, edge_index, edge_weight, W):
    raise NotImplementedError("write your pallas kernel here")


# speedup vs baseline: 1.9165x; 1.9165x over previous
import functools

import jax
import jax.numpy as jnp
from jax import lax
from jax.experimental import pallas as pl
from jax.experimental.pallas import tpu as pltpu
from jax.experimental.pallas import tpu_sc as plsc

N = 10000
E = 160000
D = 256
DQ = 64
VPE = DQ // 16
NS = 16
EPT = E // NS
K = 80
NCH = EPT // K
SLAB = 624
LAST = N - 15 * SLAB
ZR = 160


def _mm_kernel(h_ref, w_ref, o0_ref, o1_ref, o2_ref, o3_ref):
    h = h_ref[...]
    w = w_ref[...]
    for q, o_ref in enumerate((o0_ref, o1_ref, o2_ref, o3_ref)):
        o_ref[...] = jnp.dot(h, w[:, q * DQ:(q + 1) * DQ],
                             preferred_element_type=jnp.float32)


def _matmul_split(H, W):
    RB = 2000
    qshape = jax.ShapeDtypeStruct((N, DQ), jnp.float32)
    return pl.pallas_call(
        _mm_kernel,
        grid=(N // RB,),
        in_specs=[
            pl.BlockSpec((RB, D), lambda r: (r, 0)),
            pl.BlockSpec((D, D), lambda r: (0, 0)),
        ],
        out_specs=[pl.BlockSpec((RB, DQ), lambda r: (r, 0))] * 4,
        out_shape=[qshape] * 4,
    )(H, W)


def _sc_body(zero_hbm, src_hbm, dst_hbm, wgt_hbm,
             hw0_hbm, hw1_hbm, hw2_hbm, hw3_hbm,
             out0_hbm, out1_hbm, out2_hbm, out3_hbm,
             srcv, dstv, gbuf, wgtv, acc, gsem):
    c = lax.axis_index("c")
    s = lax.axis_index("s")

    pltpu.sync_copy(src_hbm.at[s], srcv)
    pltpu.sync_copy(dst_hbm.at[s], dstv)
    pltpu.sync_copy(wgt_hbm.at[s], wgtv)

    def run(hw_hbm, out_hbm):
        @pl.when(s < 15)
        def _():
            pltpu.sync_copy(zero_hbm.at[pl.ds(0, SLAB)],
                            acc.at[pl.ds(s * SLAB, SLAB)])

        @pl.when(s == 15)
        def _():
            pltpu.sync_copy(zero_hbm, acc.at[pl.ds(15 * SLAB, LAST)])

        plsc.subcore_barrier()

        def chunk(j, carry):
            pltpu.async_copy(hw_hbm.at[srcv.at[j]], gbuf, gsem).wait()

            def group(g, cc):
                wv = wgtv[j, pl.ds(g * 16, 16)]
                for lane in range(16):
                    w = wv[lane]
                    e = g * 16 + lane
                    for v in range(VPE):
                        sl = pl.ds(v * 16, 16)
                        gbuf[e, sl] = gbuf[e, sl] * w
                return cc

            lax.fori_loop(0, K // 16, group, 0)
            pltpu.sync_copy(gbuf, acc.at[dstv.at[j]], add=True)
            return carry

        lax.fori_loop(0, NCH, chunk, 0)
        plsc.subcore_barrier()

        @pl.when(s < 15)
        def _():
            pltpu.sync_copy(acc.at[pl.ds(s * SLAB, SLAB)],
                            out_hbm.at[pl.ds(s * SLAB, SLAB)])

        @pl.when(s == 15)
        def _():
            pltpu.sync_copy(acc.at[pl.ds(15 * SLAB, LAST)],
                            out_hbm.at[pl.ds(15 * SLAB, LAST)])

        plsc.subcore_barrier()

    @pl.when(c == 0)
    def _():
        run(hw0_hbm, out0_hbm)
        run(hw1_hbm, out1_hbm)

    @pl.when(c == 1)
    def _():
        run(hw2_hbm, out2_hbm)
        run(hw3_hbm, out3_hbm)


_qshape = jax.ShapeDtypeStruct((N, DQ), jnp.float32)
_sc_gcn = functools.partial(
    pl.kernel,
    mesh=plsc.VectorSubcoreMesh(core_axis_name="c", subcore_axis_name="s"),
    out_type=[_qshape] * 4,
    scratch_types=[
        pltpu.VMEM((NCH, K), jnp.int32),
        pltpu.VMEM((NCH, K), jnp.int32),
        pltpu.VMEM((K, DQ), jnp.float32),
        pltpu.VMEM((NCH, K), jnp.float32),
        pltpu.VMEM_SHARED((N, DQ), jnp.float32),
        pltpu.SemaphoreType.DMA,
    ],
    compiler_params=pltpu.CompilerParams(use_tc_tiling_on_sc=False),
)(_sc_body)


@jax.jit
def kernel(H, edge_index, edge_weight, W):
    src = edge_index[0].astype(jnp.int32).reshape(NS, NCH, K)
    dst = edge_index[1].astype(jnp.int32).reshape(NS, NCH, K)
    wgt = edge_weight.reshape(NS, NCH, K)
    zero = jnp.zeros((LAST, DQ), jnp.float32)
    hw0, hw1, hw2, hw3 = _matmul_split(H, W)
    out0, out1, out2, out3 = _sc_gcn(zero, src, dst, wgt, hw0, hw1, hw2, hw3)
    return jnp.concatenate([out0, out1, out2, out3], axis=1)


# speedup vs baseline: 3.0305x; 1.5813x over previous
import functools

import jax
import jax.numpy as jnp
from jax import lax
from jax.experimental import pallas as pl
from jax.experimental.pallas import tpu as pltpu
from jax.experimental.pallas import tpu_sc as plsc

N = 10000
E = 160000
D = 256
DQ = 64
VPE = DQ // 16
NS = 16
EPT = E // NS
K = 80
NCH = EPT // K
SLAB = 624
LAST = N - 15 * SLAB
ZR = 160


def _mm_kernel(h_ref, w_ref, o0_ref, o1_ref, o2_ref, o3_ref):
    h = h_ref[...]
    w = w_ref[...]
    for q, o_ref in enumerate((o0_ref, o1_ref, o2_ref, o3_ref)):
        o_ref[...] = jnp.dot(h, w[:, q * DQ:(q + 1) * DQ],
                             preferred_element_type=jnp.float32)


def _matmul_split(H, W):
    RB = 2000
    qshape = jax.ShapeDtypeStruct((N, DQ), jnp.float32)
    return pl.pallas_call(
        _mm_kernel,
        grid=(N // RB,),
        in_specs=[
            pl.BlockSpec((RB, D), lambda r: (r, 0)),
            pl.BlockSpec((D, D), lambda r: (0, 0)),
        ],
        out_specs=[pl.BlockSpec((RB, DQ), lambda r: (r, 0))] * 4,
        out_shape=[qshape] * 4,
    )(H, W)


PIPE = 5
GDEP = 3


def _sc_body(zero_hbm, src_hbm, dst_hbm, wgt_hbm,
             hw0_hbm, hw1_hbm, hw2_hbm, hw3_hbm,
             out0_hbm, out1_hbm, out2_hbm, out3_hbm,
             srcv, dstv, wgtv,
             gb0, gb1, gb2, gb3, gb4,
             acc,
             gs0, gs1, gs2, gs3, gs4,
             ss0, ss1, ss2, ss3, ss4):
    c = lax.axis_index("c")
    s = lax.axis_index("s")
    GB = (gb0, gb1, gb2, gb3, gb4)
    GS = (gs0, gs1, gs2, gs3, gs4)
    SS = (ss0, ss1, ss2, ss3, ss4)

    pltpu.sync_copy(src_hbm.at[s], srcv)
    pltpu.sync_copy(dst_hbm.at[s], dstv)
    pltpu.sync_copy(wgt_hbm.at[s], wgtv)

    def run(hw_hbm, out_hbm):
        @pl.when(s < 15)
        def _():
            pltpu.sync_copy(zero_hbm.at[pl.ds(0, SLAB)],
                            acc.at[pl.ds(s * SLAB, SLAB)])

        @pl.when(s == 15)
        def _():
            pltpu.sync_copy(zero_hbm, acc.at[pl.ds(15 * SLAB, LAST)])

        plsc.subcore_barrier()

        for t in range(GDEP):
            pltpu.async_copy(hw_hbm.at[srcv.at[t]], GB[t], GS[t])

        def block(blk, carry):
            for t in range(PIPE):
                jj = blk * PIPE + t
                bg = (t + GDEP) % PIPE

                @pl.when(jj >= PIPE - GDEP)
                def _():
                    pltpu.make_async_copy(
                        GB[bg], acc.at[dstv.at[jj - (PIPE - GDEP)]],
                        SS[bg]).wait()

                @pl.when(jj < NCH - GDEP)
                def _():
                    pltpu.async_copy(hw_hbm.at[srcv.at[jj + GDEP]],
                                     GB[bg], GS[bg])

                pltpu.make_async_copy(hw_hbm.at[srcv.at[jj]],
                                      GB[t], GS[t]).wait()

                def group(g, cc):
                    wv = wgtv[jj, pl.ds(g * 16, 16)]
                    for lane in range(16):
                        w = wv[lane]
                        e = g * 16 + lane
                        for v in range(VPE):
                            sl = pl.ds(v * 16, 16)
                            GB[t][e, sl] = GB[t][e, sl] * w
                    return cc

                lax.fori_loop(0, K // 16, group, 0)
                pltpu.async_copy(GB[t], acc.at[dstv.at[jj]], SS[t],
                                 add=True)
            return carry

        lax.fori_loop(0, NCH // PIPE, block, 0)

        for jj in range(NCH - (PIPE - GDEP), NCH):
            b = jj % PIPE
            pltpu.make_async_copy(GB[b], acc.at[dstv.at[jj]], SS[b]).wait()

        plsc.subcore_barrier()

        @pl.when(s < 15)
        def _():
            pltpu.sync_copy(acc.at[pl.ds(s * SLAB, SLAB)],
                            out_hbm.at[pl.ds(s * SLAB, SLAB)])

        @pl.when(s == 15)
        def _():
            pltpu.sync_copy(acc.at[pl.ds(15 * SLAB, LAST)],
                            out_hbm.at[pl.ds(15 * SLAB, LAST)])

        plsc.subcore_barrier()

    @pl.when(c == 0)
    def _():
        run(hw0_hbm, out0_hbm)
        run(hw1_hbm, out1_hbm)

    @pl.when(c == 1)
    def _():
        run(hw2_hbm, out2_hbm)
        run(hw3_hbm, out3_hbm)


_qshape = jax.ShapeDtypeStruct((N, DQ), jnp.float32)
_sc_gcn = functools.partial(
    pl.kernel,
    mesh=plsc.VectorSubcoreMesh(core_axis_name="c", subcore_axis_name="s"),
    out_type=[_qshape] * 4,
    scratch_types=(
        [
            pltpu.VMEM((NCH, K), jnp.int32),
            pltpu.VMEM((NCH, K), jnp.int32),
            pltpu.VMEM((NCH, K), jnp.float32),
        ]
        + [pltpu.VMEM((K, DQ), jnp.float32)] * PIPE
        + [pltpu.VMEM_SHARED((N, DQ), jnp.float32)]
        + [pltpu.SemaphoreType.DMA] * (2 * PIPE)
    ),
    compiler_params=pltpu.CompilerParams(use_tc_tiling_on_sc=False),
)(_sc_body)


@jax.jit
def kernel(H, edge_index, edge_weight, W):
    src = edge_index[0].astype(jnp.int32).reshape(NS, NCH, K)
    dst = edge_index[1].astype(jnp.int32).reshape(NS, NCH, K)
    wgt = edge_weight.reshape(NS, NCH, K)
    zero = jnp.zeros((LAST, DQ), jnp.float32)
    hw0, hw1, hw2, hw3 = _matmul_split(H, W)
    out0, out1, out2, out3 = _sc_gcn(zero, src, dst, wgt, hw0, hw1, hw2, hw3)
    return jnp.concatenate([out0, out1, out2, out3], axis=1)


# speedup vs baseline: 4.1216x; 1.3600x over previous
import functools

import jax
import jax.numpy as jnp
from jax import lax
from jax.experimental import pallas as pl
from jax.experimental.pallas import tpu as pltpu
from jax.experimental.pallas import tpu_sc as plsc

N = 10000
E = 160000
D = 256
DQ = 64
VPE = DQ // 16
NS = 16
EPT = E // NS
K = 80
NCH = EPT // K
SLAB = 624
LAST = N - 15 * SLAB
ZR = 160


def _mm_kernel(h_ref, w_ref, o0_ref, o1_ref, o2_ref, o3_ref):
    h = h_ref[...]
    w = w_ref[...]
    for q, o_ref in enumerate((o0_ref, o1_ref, o2_ref, o3_ref)):
        o_ref[...] = jnp.dot(h, w[:, q * DQ:(q + 1) * DQ],
                             preferred_element_type=jnp.float32)


def _matmul_split(H, W):
    RB = 2000
    qshape = jax.ShapeDtypeStruct((N, DQ), jnp.float32)
    return pl.pallas_call(
        _mm_kernel,
        grid=(N // RB,),
        in_specs=[
            pl.BlockSpec((RB, D), lambda r: (r, 0)),
            pl.BlockSpec((D, D), lambda r: (0, 0)),
        ],
        out_specs=[pl.BlockSpec((RB, DQ), lambda r: (r, 0))] * 4,
        out_shape=[qshape] * 4,
    )(H, W)


PIPE = 5
GDEP = 3


def _sc_body(zero_hbm, src_hbm, dst_hbm, wgt_hbm,
             hw0_hbm, hw1_hbm, hw2_hbm, hw3_hbm,
             out0_hbm, out1_hbm, out2_hbm, out3_hbm,
             srcv, dstv,
             gb0, gb1, gb2, gb3, gb4,
             wb0, wb1, wb2, wb3, wb4,
             acc,
             gs0, gs1, gs2, gs3, gs4,
             ss0, ss1, ss2, ss3, ss4,
             ws0, ws1, ws2, ws3, ws4):
    c = lax.axis_index("c")
    s = lax.axis_index("s")
    GB = (gb0, gb1, gb2, gb3, gb4)
    WB = (wb0, wb1, wb2, wb3, wb4)
    GS = (gs0, gs1, gs2, gs3, gs4)
    SS = (ss0, ss1, ss2, ss3, ss4)
    WS = (ws0, ws1, ws2, ws3, ws4)

    pltpu.sync_copy(src_hbm.at[s], srcv)
    pltpu.sync_copy(dst_hbm.at[s], dstv)

    def run(hw_hbm, out_hbm):
        @pl.when(s < 15)
        def _():
            pltpu.sync_copy(zero_hbm.at[pl.ds(0, SLAB)],
                            acc.at[pl.ds(s * SLAB, SLAB)])

        @pl.when(s == 15)
        def _():
            pltpu.sync_copy(zero_hbm, acc.at[pl.ds(15 * SLAB, LAST)])

        plsc.subcore_barrier()

        for t in range(GDEP):
            pltpu.async_copy(hw_hbm.at[srcv.at[t]], GB[t], GS[t])
            pltpu.async_copy(wgt_hbm.at[s, t], WB[t], WS[t])

        def block(blk, carry):
            for t in range(PIPE):
                jj = blk * PIPE + t
                bg = (t + GDEP) % PIPE

                @pl.when(jj >= PIPE - GDEP)
                def _():
                    pltpu.make_async_copy(
                        GB[bg], acc.at[dstv.at[jj - (PIPE - GDEP)]],
                        SS[bg]).wait()

                @pl.when(jj < NCH - GDEP)
                def _():
                    pltpu.async_copy(hw_hbm.at[srcv.at[jj + GDEP]],
                                     GB[bg], GS[bg])
                    pltpu.async_copy(wgt_hbm.at[s, jj + GDEP],
                                     WB[bg], WS[bg])

                pltpu.make_async_copy(hw_hbm.at[srcv.at[jj]],
                                      GB[t], GS[t]).wait()
                pltpu.make_async_copy(wgt_hbm.at[s, jj],
                                      WB[t], WS[t]).wait()

                def edges(it, cc):
                    for kq in range(4):
                        e = it * 4 + kq
                        wvec = WB[t][e]
                        for v in range(VPE):
                            sl = pl.ds(v * 16, 16)
                            GB[t][e, sl] = GB[t][e, sl] * wvec
                    return cc

                lax.fori_loop(0, K // 4, edges, 0)
                pltpu.async_copy(GB[t], acc.at[dstv.at[jj]], SS[t],
                                 add=True)
            return carry

        lax.fori_loop(0, NCH // PIPE, block, 0)

        for jj in range(NCH - (PIPE - GDEP), NCH):
            b = jj % PIPE
            pltpu.make_async_copy(GB[b], acc.at[dstv.at[jj]], SS[b]).wait()

        plsc.subcore_barrier()

        @pl.when(s < 15)
        def _():
            pltpu.sync_copy(acc.at[pl.ds(s * SLAB, SLAB)],
                            out_hbm.at[pl.ds(s * SLAB, SLAB)])

        @pl.when(s == 15)
        def _():
            pltpu.sync_copy(acc.at[pl.ds(15 * SLAB, LAST)],
                            out_hbm.at[pl.ds(15 * SLAB, LAST)])

        plsc.subcore_barrier()

    @pl.when(c == 0)
    def _():
        run(hw0_hbm, out0_hbm)
        run(hw1_hbm, out1_hbm)

    @pl.when(c == 1)
    def _():
        run(hw2_hbm, out2_hbm)
        run(hw3_hbm, out3_hbm)


_qshape = jax.ShapeDtypeStruct((N, DQ), jnp.float32)
_sc_gcn = functools.partial(
    pl.kernel,
    mesh=plsc.VectorSubcoreMesh(core_axis_name="c", subcore_axis_name="s"),
    out_type=[_qshape] * 4,
    scratch_types=(
        [
            pltpu.VMEM((NCH, K), jnp.int32),
            pltpu.VMEM((NCH, K), jnp.int32),
        ]
        + [pltpu.VMEM((K, DQ), jnp.float32)] * PIPE
        + [pltpu.VMEM((K, 16), jnp.float32)] * PIPE
        + [pltpu.VMEM_SHARED((N, DQ), jnp.float32)]
        + [pltpu.SemaphoreType.DMA] * (3 * PIPE)
    ),
    compiler_params=pltpu.CompilerParams(use_tc_tiling_on_sc=False),
)(_sc_body)


@jax.jit
def kernel(H, edge_index, edge_weight, W):
    src = edge_index[0].astype(jnp.int32).reshape(NS, NCH, K)
    dst = edge_index[1].astype(jnp.int32).reshape(NS, NCH, K)
    wgt = jnp.broadcast_to(
        edge_weight.reshape(NS, NCH, K)[..., None], (NS, NCH, K, 16)
    )
    zero = jnp.zeros((LAST, DQ), jnp.float32)
    hw0, hw1, hw2, hw3 = _matmul_split(H, W)
    out0, out1, out2, out3 = _sc_gcn(zero, src, dst, wgt, hw0, hw1, hw2, hw3)
    return jnp.concatenate([out0, out1, out2, out3], axis=1)
